# baseline (speedup 1.0000x reference)
import functools

import jax
import jax.numpy as jnp
from jax import lax
from jax.experimental import pallas as pl
from jax.experimental.pallas import tpu as pltpu
from jax.experimental.pallas import tpu_sc as plsc

B = 32
N = 512 * 512
NC, NS, L = 2, 16, 16
NW = NC * NS

SC_B = 8
K = NW // SC_B
WROWS = 512 // K
SLAB = 32
NSLAB = WROWS // SLAB
ROWV = 512 // L

SPB = 4
TC_B = B - SC_B
TC_GRID = TC_B // SPB


def _log1p_poly(z):
    u = z * z
    p = 2.0 / 9.0 + u * (2.0 / 11.0)
    p = 2.0 / 7.0 + u * p
    p = 2.0 / 5.0 + u * p
    p = 2.0 / 3.0 + u * p
    p = 2.0 + u * p
    return z * p


def _sc_body(x_hbm, t_hbm, out_hbm, xbuf, tbuf, obuf):
    c = lax.axis_index("c")
    s = lax.axis_index("s")
    w = s * NC + c
    sid = w // K
    row0 = (w % K) * WROWS

    zero = jnp.zeros((L,), jnp.float32)
    acc = (zero, zero, zero, zero)

    def inner(r, carry):
        sacc, stacc, tacc, bacc = carry
        for u in range(ROWV):
            xv = xbuf[r, pl.ds(u * L, L)]
            tv = tbuf[r, pl.ds(u * L, L)].astype(jnp.float32)
            ax = jnp.abs(xv)
            e = jnp.exp(-ax)
            a = 1.0 + e
            b = 2.0 + e
            q = 1.0 / (a * b)
            inv = q * b
            z = (e * q) * a
            sig = jnp.where(xv >= 0.0, inv, e * inv)
            bce = jnp.maximum(xv, 0.0) - xv * tv + _log1p_poly(z)
            sacc = sacc + sig
            stacc = stacc + sig * tv
            tacc = tacc + tv
            bacc = bacc + bce
        return (sacc, stacc, tacc, bacc)

    for slab in range(NSLAB):
        pltpu.sync_copy(x_hbm.at[sid, pl.ds(row0 + slab * SLAB, SLAB)], xbuf)
        pltpu.sync_copy(t_hbm.at[sid, pl.ds(row0 + slab * SLAB, SLAB)], tbuf)
        acc = lax.fori_loop(0, SLAB, inner, acc)

    for k in range(4):
        obuf[pl.ds(k * L, L)] = acc[k]
    pltpu.sync_copy(obuf, out_hbm.at[w])


_sc_partials = functools.partial(
    pl.kernel,
    out_type=jax.ShapeDtypeStruct((NW, 128), jnp.float32),
    mesh=plsc.VectorSubcoreMesh(
        core_axis_name="c", subcore_axis_name="s",
        num_cores=NC, num_subcores=NS),
    scratch_types=[
        pltpu.VMEM((SLAB, 512), jnp.float32),
        pltpu.VMEM((SLAB, 512), jnp.int32),
        pltpu.VMEM((128,), jnp.float32),
    ],
    compiler_params=pltpu.CompilerParams(use_tc_tiling_on_sc=True),
)(_sc_body)


def _tc_body(x_ref, t_ref, out_ref):
    x = x_ref[...].reshape(SPB, N)
    t = t_ref[...].reshape(SPB, N).astype(jnp.float32)

    ax = jnp.abs(x)
    e = jnp.exp(-ax)
    inv = 1.0 / (1.0 + e)
    sig = jnp.where(x >= 0.0, inv, e * inv)
    bce = jnp.maximum(x, 0.0) - x * t + jnp.log1p(e)

    s_sum = jnp.sum(sig, axis=1)
    st_sum = jnp.sum(sig * t, axis=1)
    t_sum = jnp.sum(t, axis=1)
    b_sum = jnp.sum(bce, axis=1)

    col = lax.broadcasted_iota(jnp.int32, (1, SPB, 128), 2)
    out_ref[...] = (jnp.where(col == 0, s_sum[None, :, None], 0.0)
                    + jnp.where(col == 1, st_sum[None, :, None], 0.0)
                    + jnp.where(col == 2, t_sum[None, :, None], 0.0)
                    + jnp.where(col == 3, b_sum[None, :, None], 0.0))


def _tc_partials(inputs, target):
    return pl.pallas_call(
        _tc_body,
        grid=(TC_GRID,),
        in_specs=[
            pl.BlockSpec((SPB, 1, 512, 512), lambda i: (SC_B // SPB + i, 0, 0, 0)),
            pl.BlockSpec((SPB, 512, 512), lambda i: (SC_B // SPB + i, 0, 0)),
        ],
        out_specs=pl.BlockSpec((1, SPB, 128), lambda i: (i, 0, 0)),
        out_shape=jax.ShapeDtypeStruct((TC_GRID, SPB, 128), jnp.float32),
        compiler_params=pltpu.CompilerParams(
            dimension_semantics=("arbitrary",),
        ),
    )(inputs, target)


def _tc_all(inputs, target):
    return pl.pallas_call(
        _tc_body,
        grid=(B // SPB,),
        in_specs=[
            pl.BlockSpec((SPB, 1, 512, 512), lambda i: (i, 0, 0, 0)),
            pl.BlockSpec((SPB, 512, 512), lambda i: (i, 0, 0)),
        ],
        out_specs=pl.BlockSpec((1, SPB, 128), lambda i: (i, 0, 0)),
        out_shape=jax.ShapeDtypeStruct((B // SPB, SPB, 128), jnp.float32),
        compiler_params=pltpu.CompilerParams(
            dimension_semantics=("arbitrary",),
        ),
    )(inputs, target)


@jax.jit
def kernel(inputs, target):
    parts = _tc_all(inputs, target).reshape(B, 128)[:, :4]
    s_sum = parts[:, 0]
    st_sum = parts[:, 1]
    t_sum = parts[:, 2]
    b_sum = parts[:, 3]
    dice = 1.0 - (2.0 * st_sum + 1.0) / (s_sum + t_sum + 1.0)
    loss = jnp.mean(dice) + jnp.sum(b_sum) / (B * N)
    return loss.reshape(1)


# speedup vs baseline: 1.3967x; 1.3967x over previous
import functools

import jax
import jax.numpy as jnp
from jax import lax
from jax.experimental import pallas as pl
from jax.experimental.pallas import tpu as pltpu
from jax.experimental.pallas import tpu_sc as plsc

B = 32
N = 512 * 512
NC, NS, L = 2, 16, 16
NW = NC * NS

SC_B = 8
K = NW // SC_B
WROWS = 512 // K
SLAB = 32
NSLAB = WROWS // SLAB
ROWV = 512 // L

SPB = 4
TC_B = B - SC_B
TC_GRID = TC_B // SPB


def _log1p_poly(z):
    u = z * z
    p = 2.0 / 9.0 + u * (2.0 / 11.0)
    p = 2.0 / 7.0 + u * p
    p = 2.0 / 5.0 + u * p
    p = 2.0 / 3.0 + u * p
    p = 2.0 + u * p
    return z * p


def _sc_body(x_hbm, t_hbm, out_hbm, xbuf, tbuf, obuf):
    c = lax.axis_index("c")
    s = lax.axis_index("s")
    w = s * NC + c
    sid = w // K
    row0 = (w % K) * WROWS

    zero = jnp.zeros((L,), jnp.float32)
    acc = (zero, zero, zero, zero)

    def inner(r, carry):
        sacc, stacc, tacc, bacc = carry
        for u in range(ROWV):
            xv = xbuf[r, pl.ds(u * L, L)]
            tv = tbuf[r, pl.ds(u * L, L)].astype(jnp.float32)
            ax = jnp.abs(xv)
            e = jnp.exp(-ax)
            a = 1.0 + e
            b = 2.0 + e
            q = 1.0 / (a * b)
            inv = q * b
            z = (e * q) * a
            sig = jnp.where(xv >= 0.0, inv, e * inv)
            bce = jnp.maximum(xv, 0.0) - xv * tv + _log1p_poly(z)
            sacc = sacc + sig
            stacc = stacc + sig * tv
            tacc = tacc + tv
            bacc = bacc + bce
        return (sacc, stacc, tacc, bacc)

    for slab in range(NSLAB):
        pltpu.sync_copy(x_hbm.at[sid, pl.ds(row0 + slab * SLAB, SLAB)], xbuf)
        pltpu.sync_copy(t_hbm.at[sid, pl.ds(row0 + slab * SLAB, SLAB)], tbuf)
        acc = lax.fori_loop(0, SLAB, inner, acc)

    for k in range(4):
        obuf[pl.ds(k * L, L)] = acc[k]
    pltpu.sync_copy(obuf, out_hbm.at[w])


_sc_partials = functools.partial(
    pl.kernel,
    out_type=jax.ShapeDtypeStruct((NW, 128), jnp.float32),
    mesh=plsc.VectorSubcoreMesh(
        core_axis_name="c", subcore_axis_name="s",
        num_cores=NC, num_subcores=NS),
    scratch_types=[
        pltpu.VMEM((SLAB, 512), jnp.float32),
        pltpu.VMEM((SLAB, 512), jnp.int32),
        pltpu.VMEM((128,), jnp.float32),
    ],
    compiler_params=pltpu.CompilerParams(use_tc_tiling_on_sc=True),
)(_sc_body)


def _tc_body(x_ref, t_ref, out_ref):
    i = pl.program_id(0)
    x = x_ref[...]
    t = t_ref[...].astype(jnp.float32)

    ax = jnp.abs(x)
    e = jnp.exp(-ax)
    inv = 1.0 / (1.0 + e)
    sig = jnp.where(x >= 0.0, inv, e * inv)
    bce = jnp.maximum(x, 0.0) - x * t + jnp.log1p(e)

    for u in range(SPB):
        out_ref[i * SPB + u, 0] = jnp.sum(sig[u])
        out_ref[i * SPB + u, 1] = jnp.sum(sig[u] * t[u])
        out_ref[i * SPB + u, 2] = jnp.sum(t[u])
        out_ref[i * SPB + u, 3] = jnp.sum(bce[u])


def _tc_partials(x3, target):
    return pl.pallas_call(
        _tc_body,
        grid=(TC_GRID,),
        in_specs=[
            pl.BlockSpec((SPB, 512, 512), lambda i: (SC_B // SPB + i, 0, 0)),
            pl.BlockSpec((SPB, 512, 512), lambda i: (SC_B // SPB + i, 0, 0)),
        ],
        out_specs=pl.BlockSpec(memory_space=pltpu.SMEM),
        out_shape=jax.ShapeDtypeStruct((TC_B, 4), jnp.float32),
        compiler_params=pltpu.CompilerParams(
            dimension_semantics=("arbitrary",),
        ),
    )(x3, target)


@jax.jit
def kernel(inputs, target):
    x3 = inputs.reshape(B, 512, 512)
    sc_w = _sc_partials(x3, target)[:, :64].reshape(NW, 4, L).sum(axis=2)
    sc_parts = sc_w.reshape(SC_B, K, 4).sum(axis=1)
    tc_parts = _tc_partials(x3, target)
    parts = jnp.concatenate([sc_parts, tc_parts], axis=0)
    s_sum = parts[:, 0]
    st_sum = parts[:, 1]
    t_sum = parts[:, 2]
    b_sum = parts[:, 3]
    dice = 1.0 - (2.0 * st_sum + 1.0) / (s_sum + t_sum + 1.0)
    loss = jnp.mean(dice) + jnp.sum(b_sum) / (B * N)
    return loss.reshape(1)


# speedup vs baseline: 1.4367x; 1.0286x over previous
import functools

import jax
import jax.numpy as jnp
from jax import lax
from jax.experimental import pallas as pl
from jax.experimental.pallas import tpu as pltpu
from jax.experimental.pallas import tpu_sc as plsc

B = 32
N = 512 * 512
NC, NS, L = 2, 16, 16
NW = NC * NS

SC_B = 8
K = NW // SC_B
WROWS = 512 // K
SLAB = 32
NSLAB = WROWS // SLAB
ROWV = 512 // L

SPB = 4
TC_B = B - SC_B
TC_GRID = TC_B // SPB


def _log1p_poly(z):
    u = z * z
    p = 2.0 / 5.0 + u * (2.0 / 7.0)
    p = 2.0 / 3.0 + u * p
    p = 2.0 + u * p
    return z * p


def _sc_body(x_hbm, t_hbm, out_hbm, xb0, xb1, tb0, tb1, obuf, sem0, sem1):
    c = lax.axis_index("c")
    s = lax.axis_index("s")
    w = s * NC + c
    sid = w // K
    row0 = (w % K) * WROWS

    xbufs = (xb0, xb1)
    tbufs = (tb0, tb1)
    sems = (sem0, sem1)

    def start(slab, p):
        hx = pltpu.async_copy(
            x_hbm.at[sid, pl.ds(row0 + slab * SLAB, SLAB)], xbufs[p], sems[p])
        ht = pltpu.async_copy(
            t_hbm.at[sid, pl.ds(row0 + slab * SLAB, SLAB)], tbufs[p], sems[p])
        return hx, ht

    zero = jnp.zeros((L,), jnp.float32)
    acc = (zero, zero, zero, zero)
    pend = start(0, 0)

    for slab in range(NSLAB):
        p = slab % 2
        nxt = start(slab + 1, 1 - p) if slab + 1 < NSLAB else None
        pend[0].wait()
        pend[1].wait()
        xbuf, tbuf = xbufs[p], tbufs[p]

        @plsc.parallel_loop(0, SLAB, unroll=2, carry=acc)
        def inner(r, carry, xbuf=xbuf, tbuf=tbuf):
            sacc, stacc, tacc, bacc = carry
            for u in range(ROWV):
                xv = xbuf[r, pl.ds(u * L, L)]
                tv = tbuf[r, pl.ds(u * L, L)].astype(jnp.float32)
                ax = jnp.abs(xv)
                e = jnp.exp(-ax)
                a = 1.0 + e
                b = a + 1.0
                q = 1.0 / (a * b)
                inv = q * b
                z = (e * q) * a
                sig = jnp.where(xv >= 0.0, inv, e * inv)
                bce = jnp.maximum(xv, 0.0) - xv * tv + _log1p_poly(z)
                sacc = sacc + sig
                stacc = stacc + sig * tv
                tacc = tacc + tv
                bacc = bacc + bce
            return (sacc, stacc, tacc, bacc)

        acc = inner
        pend = nxt

    for k in range(4):
        obuf[pl.ds(k * L, L)] = acc[k]
    pltpu.sync_copy(obuf, out_hbm.at[w])


_sc_partials = functools.partial(
    pl.kernel,
    out_type=jax.ShapeDtypeStruct((NW, 128), jnp.float32),
    mesh=plsc.VectorSubcoreMesh(
        core_axis_name="c", subcore_axis_name="s",
        num_cores=NC, num_subcores=NS),
    scratch_types=[
        pltpu.VMEM((SLAB, 512), jnp.float32),
        pltpu.VMEM((SLAB, 512), jnp.float32),
        pltpu.VMEM((SLAB, 512), jnp.int32),
        pltpu.VMEM((SLAB, 512), jnp.int32),
        pltpu.VMEM((128,), jnp.float32),
        pltpu.SemaphoreType.DMA,
        pltpu.SemaphoreType.DMA,
    ],
    compiler_params=pltpu.CompilerParams(use_tc_tiling_on_sc=True),
)(_sc_body)


def _tc_body(x_ref, t_ref, out_ref):
    i = pl.program_id(0)
    x = x_ref[...]
    t = t_ref[...].astype(jnp.float32)

    ax = jnp.abs(x)
    e = jnp.exp(-ax)
    inv = 1.0 / (1.0 + e)
    sig = jnp.where(x >= 0.0, inv, e * inv)
    bce = jnp.maximum(x, 0.0) - x * t + jnp.log1p(e)

    for u in range(SPB):
        out_ref[i * SPB + u, 0] = jnp.sum(sig[u])
        out_ref[i * SPB + u, 1] = jnp.sum(sig[u] * t[u])
        out_ref[i * SPB + u, 2] = jnp.sum(t[u])
        out_ref[i * SPB + u, 3] = jnp.sum(bce[u])


def _tc_partials(x3, target):
    return pl.pallas_call(
        _tc_body,
        grid=(TC_GRID,),
        in_specs=[
            pl.BlockSpec((SPB, 512, 512), lambda i: (SC_B // SPB + i, 0, 0)),
            pl.BlockSpec((SPB, 512, 512), lambda i: (SC_B // SPB + i, 0, 0)),
        ],
        out_specs=pl.BlockSpec(memory_space=pltpu.SMEM),
        out_shape=jax.ShapeDtypeStruct((TC_B, 4), jnp.float32),
        compiler_params=pltpu.CompilerParams(
            dimension_semantics=("arbitrary",),
        ),
    )(x3, target)


@jax.jit
def kernel(inputs, target):
    x3 = inputs.reshape(B, 512, 512)
    sc_w = _sc_partials(x3, target)[:, :64].reshape(NW, 4, L).sum(axis=2)
    sc_parts = sc_w.reshape(SC_B, K, 4).sum(axis=1)
    tc_parts = _tc_partials(x3, target)
    parts = jnp.concatenate([sc_parts, tc_parts], axis=0)
    s_sum = parts[:, 0]
    st_sum = parts[:, 1]
    t_sum = parts[:, 2]
    b_sum = parts[:, 3]
    dice = 1.0 - (2.0 * st_sum + 1.0) / (s_sum + t_sum + 1.0)
    loss = jnp.mean(dice) + jnp.sum(b_sum) / (B * N)
    return loss.reshape(1)


# speedup vs baseline: 1.6276x; 1.1329x over previous
import jax
import jax.numpy as jnp
from jax import lax
from jax.experimental import pallas as pl
from jax.experimental.pallas import tpu as pltpu

B = 32
N = 512 * 512
SPB = 4
GRID = B // SPB


def _tc_body(x_ref, t_ref, s_ref, st_ref, t_sum_ref, b_ref):
    x = x_ref[...]
    t = t_ref[...].astype(jnp.float32)

    ax = jnp.abs(x)
    e = jnp.exp(-ax)
    inv = 1.0 / (1.0 + e)
    sig = jnp.where(x >= 0.0, inv, 1.0 - inv)
    bce = jnp.maximum(x, 0.0) - x * t + jnp.log1p(e)

    s_ref[0] = jnp.sum(sig, axis=1)
    st_ref[0] = jnp.sum(sig * t, axis=1)
    t_sum_ref[0] = jnp.sum(t, axis=1)
    b_ref[0] = jnp.sum(bce, axis=1)


def _tc_partials(x3, target):
    part = pl.BlockSpec((1, SPB, 512), lambda i: (i, 0, 0))
    shape = jax.ShapeDtypeStruct((GRID, SPB, 512), jnp.float32)
    return pl.pallas_call(
        _tc_body,
        grid=(GRID,),
        in_specs=[
            pl.BlockSpec((SPB, 512, 512), lambda i: (i, 0, 0)),
            pl.BlockSpec((SPB, 512, 512), lambda i: (i, 0, 0)),
        ],
        out_specs=(part, part, part, part),
        out_shape=(shape, shape, shape, shape),
        compiler_params=pltpu.CompilerParams(
            dimension_semantics=("arbitrary",),
        ),
    )(x3, target)


@jax.jit
def kernel(inputs, target):
    x3 = inputs.reshape(B, 512, 512)
    s_p, st_p, t_p, b_p = _tc_partials(x3, target)
    s_sum = s_p.reshape(B, 512).sum(axis=1)
    st_sum = st_p.reshape(B, 512).sum(axis=1)
    t_sum = t_p.reshape(B, 512).sum(axis=1)
    b_sum = b_p.reshape(B, 512).sum(axis=1)
    dice = 1.0 - (2.0 * st_sum + 1.0) / (s_sum + t_sum + 1.0)
    loss = jnp.mean(dice) + jnp.sum(b_sum) / (B * N)
    return loss.reshape(1)


# speedup vs baseline: 2.3435x; 1.4399x over previous
import jax
import jax.numpy as jnp
from jax import lax
from jax.experimental import pallas as pl
from jax.experimental.pallas import tpu as pltpu

B = 32
N = 512 * 512
SPB = 4
GRID = B // SPB


def _tc_body(x_ref, t_ref, out_ref):
    i = pl.program_id(0)
    x = x_ref[...]
    t = t_ref[...].astype(jnp.float32)

    ax = jnp.abs(x)
    e = jnp.exp(-ax)
    inv = 1.0 / (1.0 + e)
    sig = jnp.where(x >= 0.0, inv, 1.0 - inv)
    bce = jnp.maximum(x, 0.0) - x * t + jnp.log1p(e)

    for u in range(SPB):
        out_ref[i * SPB + u, 0] = jnp.sum(sig[u])
        out_ref[i * SPB + u, 1] = jnp.sum(sig[u] * t[u])
        out_ref[i * SPB + u, 2] = jnp.sum(t[u])
        out_ref[i * SPB + u, 3] = jnp.sum(bce[u])


def _tc_partials(x3, target):
    return pl.pallas_call(
        _tc_body,
        grid=(GRID,),
        in_specs=[
            pl.BlockSpec((SPB, 512, 512), lambda i: (i, 0, 0)),
            pl.BlockSpec((SPB, 512, 512), lambda i: (i, 0, 0)),
        ],
        out_specs=pl.BlockSpec(memory_space=pltpu.SMEM),
        out_shape=jax.ShapeDtypeStruct((B, 4), jnp.float32),
        compiler_params=pltpu.CompilerParams(
            dimension_semantics=("arbitrary",),
        ),
    )(x3, target)


@jax.jit
def kernel(inputs, target):
    x3 = inputs.reshape(B, 512, 512)
    parts = _tc_partials(x3, target)
    s_sum = parts[:, 0]
    st_sum = parts[:, 1]
    t_sum = parts[:, 2]
    b_sum = parts[:, 3]
    dice = 1.0 - (2.0 * st_sum + 1.0) / (s_sum + t_sum + 1.0)
    loss = jnp.mean(dice) + jnp.sum(b_sum) / (B * N)
    return loss.reshape(1)
